# baseline (speedup 1.0000x reference)
import functools

import jax
import jax.numpy as jnp
from jax import lax
from jax.experimental import pallas as pl
from jax.experimental.pallas import tpu as pltpu
from jax.experimental.pallas import tpu_sc as plsc

HIGHEST = lax.Precision.HIGHEST

_NC = 2
_NS = 16
_NW = _NC * _NS


def _f32dot(a, b):
    return jnp.dot(a, b, preferred_element_type=jnp.float32)


def _embed_call(N, H, R, AP):
    nb = N // R

    def body(h_ref, tab_ref, x_ref, s_ref):
        hv = h_ref[...]
        iot = lax.broadcasted_iota(jnp.int32, (R, AP), 1)
        oh = jnp.where(hv == iot, 1.0, 0.0).astype(jnp.float32)
        x = _f32dot(oh, tab_ref[...])
        x_ref[...] = x
        col = jnp.sum(x, axis=0, keepdims=True)

        @pl.when(pl.program_id(0) == 0)
        def _():
            s_ref[...] = col

        @pl.when(pl.program_id(0) > 0)
        def _():
            s_ref[...] += col

    return pl.pallas_call(
        body,
        grid=(nb,),
        in_specs=[
            pl.BlockSpec((R, 1), lambda i: (i, 0)),
            pl.BlockSpec((AP, H), lambda i: (0, 0)),
        ],
        out_specs=[
            pl.BlockSpec((R, H), lambda i: (i, 0)),
            pl.BlockSpec((1, H), lambda i: (0, 0)),
        ],
        out_shape=[
            jax.ShapeDtypeStruct((N, H), jnp.float32),
            jax.ShapeDtypeStruct((1, H), jnp.float32),
        ],
    )


def _layer_call(N, H, R, first, last):
    def body(*refs):
        if first:
            (h_ref, aa_ref, ab_ref, da_ref, db_ref, ws_ref, wn_ref, b_ref,
             o_ref, dg_ref, s_ref) = refs
            d = jnp.maximum(da_ref[...][:, 0:1] + db_ref[...][:, 0:1], 1.0)
            dg_ref[...] = d
        elif last:
            (h_ref, aa_ref, ab_ref, dg_ref, ws_ref, wn_ref, b_ref,
             s_ref) = refs
            d = dg_ref[...]
        else:
            (h_ref, aa_ref, ab_ref, dg_ref, ws_ref, wn_ref, b_ref,
             o_ref, s_ref) = refs
            d = dg_ref[...]
        agg = (aa_ref[...] + ab_ref[...]) / d
        hn = jnp.maximum(
            _f32dot(h_ref[...], ws_ref[...]) + _f32dot(agg, wn_ref[...])
            + b_ref[...], 0.0)
        if not last:
            o_ref[...] = hn
        col = jnp.sum(hn, axis=0, keepdims=True)

        @pl.when(pl.program_id(0) == 0)
        def _():
            s_ref[...] = col

        @pl.when(pl.program_id(0) > 0)
        def _():
            s_ref[...] += col

    in_specs = [
        pl.BlockSpec((R, H), lambda i: (i, 0)),
        pl.BlockSpec((R, H), lambda i: (i, 0)),
        pl.BlockSpec((R, H), lambda i: (i, 0)),
    ]
    if first:
        in_specs += [
            pl.BlockSpec((R, H), lambda i: (i, 0)),
            pl.BlockSpec((R, H), lambda i: (i, 0)),
        ]
    else:
        in_specs += [pl.BlockSpec((R, 1), lambda i: (i, 0))]
    in_specs += [pl.BlockSpec((H, H), lambda i: (0, 0)),
                 pl.BlockSpec((H, H), lambda i: (0, 0)),
                 pl.BlockSpec((1, H), lambda i: (0, 0))]

    out_specs, out_shape = [], []
    if not last:
        out_specs += [pl.BlockSpec((R, H), lambda i: (i, 0))]
        out_shape += [jax.ShapeDtypeStruct((N, H), jnp.float32)]
    if first:
        out_specs += [pl.BlockSpec((R, 1), lambda i: (i, 0))]
        out_shape += [jax.ShapeDtypeStruct((N, 1), jnp.float32)]
    out_specs += [pl.BlockSpec((1, H), lambda i: (0, 0))]
    out_shape += [jax.ShapeDtypeStruct((1, H), jnp.float32)]

    return pl.pallas_call(
        body, grid=(N // R,), in_specs=in_specs, out_specs=out_specs,
        out_shape=out_shape,
    )


def _readout_call(N, H, D):
    inv_n = 1.0 / N

    def body(s0, s1, s2, s3, w1, b1, w2, b2, w3, b3, o_ref):
        acc = b1[...]
        for k, s in enumerate((s0, s1, s2, s3)):
            acc = acc + _f32dot(s[...] * inv_n, w1[k * H:(k + 1) * H, :])
        o1 = jnp.maximum(acc, 0.0)
        o2 = jnp.maximum(_f32dot(o1, w2[...]) + b2[...], 0.0)
        o_ref[...] = _f32dot(o2, w3[...]) + b3[...]

    return pl.pallas_call(
        body,
        out_shape=jax.ShapeDtypeStruct((1, 1), jnp.float32),
    )


_C = 64
_KI = 16
_NP = 10016
_SC0_SHARE_PCT = 35


def _agg_call(N, NCH):
    H = 128
    KB = NCH // _KI
    RA = ((_NP // _NS) // 8) * 8
    RL = _NP - (_NS - 1) * RA
    assert KB * _KI == NCH and RL % 8 == 0 and RL >= RA

    mesh = plsc.VectorSubcoreMesh(core_axis_name="c", subcore_axis_name="s")

    def chunks(total):
        out, ofs = [], 0
        while ofs < total:
            c = min(_C, total - ofs)
            out.append((ofs, c))
            ofs += c
        return out

    out_type = tuple(
        jax.ShapeDtypeStruct((_NP, H), jnp.float32) for _ in range(2))
    scratch = [
        pltpu.VMEM((_KI, _C), jnp.int32),
        pltpu.VMEM((_KI, _C), jnp.int32),
        pltpu.VMEM((_C, H), jnp.float32),
        pltpu.VMEM((_C, H), jnp.float32),
        pltpu.VMEM_SHARED((_NP, H), jnp.float32),
        pltpu.SemaphoreType.DMA,
        pltpu.SemaphoreType.DMA,
    ]

    KB0 = (2 * KB * _SC0_SHARE_PCT + 50) // 100
    KB1 = 2 * KB - KB0

    def body(p_hbm, srcr, dstr, zrow, oa0, oa1,
             sidx, didx, rows0, rows1, accs, sem0, sem1):
        cid = lax.axis_index("c")
        sid = lax.axis_index("s")
        base = pl.multiple_of(sid * RA, 8)
        kb_loc = jnp.where(cid == 0, KB0, KB1)
        irow0 = pl.multiple_of(
            jnp.where(cid == 0, sid * (KB0 * _KI),
                      _NS * (KB0 * _KI) + sid * (KB1 * _KI)), 8)

        pltpu.sync_copy(zrow, rows0)
        for is_last, rcnt in ((False, RA), (True, RL)):
            cond = (sid == _NS - 1) if is_last else (sid < _NS - 1)

            @pl.when(cond)
            def _():
                for ofs, c in chunks(rcnt):
                    sl = pl.ds(pl.multiple_of(base + ofs, 8), c)
                    pltpu.sync_copy(rows0.at[pl.ds(0, c)], accs.at[sl])

        plsc.subcore_barrier()

        def outer(kb, carry):
            koff = pl.multiple_of(irow0 + kb * _KI, 8)
            pltpu.sync_copy(srcr.at[pl.ds(koff, _KI)], sidx)
            pltpu.sync_copy(dstr.at[pl.ds(koff, _KI)], didx)
            pltpu.make_async_copy(p_hbm.at[sidx.at[0]], rows0, sem0).start()

            def pipe(j, cur, csem, nxt, nsem):
                pltpu.make_async_copy(p_hbm.at[sidx.at[j]], cur, csem).wait()

                @pl.when(j < _KI - 1)
                def _():
                    pltpu.make_async_copy(
                        p_hbm.at[sidx.at[j + 1]], nxt, nsem).start()

                pltpu.sync_copy(cur, accs.at[didx.at[j]], add=True)

            def step(j, c2):
                even = j % 2 == 0

                @pl.when(even)
                def _():
                    pipe(j, rows0, sem0, rows1, sem1)

                @pl.when(jnp.logical_not(even))
                def _():
                    pipe(j, rows1, sem1, rows0, sem0)

                return c2

            lax.fori_loop(0, _KI, step, 0)
            return carry

        lax.fori_loop(0, kb_loc, outer, 0)
        plsc.subcore_barrier()

        for is_last, rcnt in ((False, RA), (True, RL)):
            cond = (sid == _NS - 1) if is_last else (sid < _NS - 1)

            @pl.when(cond)
            def _():
                for c0, oa in enumerate((oa0, oa1)):
                    @pl.when(cid == c0)
                    def _():
                        for ofs, c in chunks(rcnt):
                            sl = pl.ds(pl.multiple_of(base + ofs, 8), c)
                            pltpu.sync_copy(accs.at[sl], rows0.at[pl.ds(0, c)])
                            pltpu.sync_copy(rows0.at[pl.ds(0, c)], oa.at[sl])

    return pl.kernel(
        body, mesh=mesh, out_type=out_type, scratch_types=scratch,
    )


def _deg_call(N, NCH):
    H = 128
    KB = NCH // _KI
    RA = ((_NP // _NS) // 8) * 8
    RL = _NP - (_NS - 1) * RA

    mesh = plsc.VectorSubcoreMesh(core_axis_name="c", subcore_axis_name="s")

    def chunks(total):
        out, ofs = [], 0
        while ofs < total:
            c = min(_C, total - ofs)
            out.append((ofs, c))
            ofs += c
        return out

    out_type = tuple(
        jax.ShapeDtypeStruct((_NP, H), jnp.float32) for _ in range(2))
    scratch = [
        pltpu.VMEM((_KI, _C), jnp.int32),
        pltpu.VMEM((_C, H), jnp.float32),
        pltpu.VMEM_SHARED((_NP, H), jnp.float32),
    ]

    def body(ones_hbm, dstr, zrow, od0, od1, didx, rows, accs):
        cid = lax.axis_index("c")
        sid = lax.axis_index("s")
        base = pl.multiple_of(sid * RA, 8)
        irow0 = pl.multiple_of((cid * _NS + sid) * NCH, 8)

        pltpu.sync_copy(zrow, rows)
        for is_last, rcnt in ((False, RA), (True, RL)):
            cond = (sid == _NS - 1) if is_last else (sid < _NS - 1)

            @pl.when(cond)
            def _():
                for ofs, c in chunks(rcnt):
                    sl = pl.ds(pl.multiple_of(base + ofs, 8), c)
                    pltpu.sync_copy(rows.at[pl.ds(0, c)], accs.at[sl])

        pltpu.sync_copy(ones_hbm, rows)
        plsc.subcore_barrier()

        def outer(kb, carry):
            koff = pl.multiple_of(irow0 + kb * _KI, 8)
            pltpu.sync_copy(dstr.at[pl.ds(koff, _KI)], didx)

            def step(j, c2):
                pltpu.sync_copy(rows, accs.at[didx.at[j]], add=True)
                return c2

            lax.fori_loop(0, _KI, step, 0)
            return carry

        lax.fori_loop(0, KB, outer, 0)
        plsc.subcore_barrier()

        for is_last, rcnt in ((False, RA), (True, RL)):
            cond = (sid == _NS - 1) if is_last else (sid < _NS - 1)

            @pl.when(cond)
            def _():
                for c0, od in enumerate((od0, od1)):
                    @pl.when(cid == c0)
                    def _():
                        for ofs, c in chunks(rcnt):
                            sl = pl.ds(pl.multiple_of(base + ofs, 8), c)
                            pltpu.sync_copy(accs.at[sl], rows.at[pl.ds(0, c)])
                            pltpu.sync_copy(rows.at[pl.ds(0, c)], od.at[sl])

    return pl.kernel(
        body, mesh=mesh, out_type=out_type, scratch_types=scratch,
    )


def kernel(h, edge_index, e, embed_table, W_self, W_neigh, b_layers,
           Wr1, br1, Wr2, br2, Wr3, br3):
    del e
    N = h.shape[0]
    E = edge_index.shape[1]
    A, H = embed_table.shape
    AP = 128
    R = 2000

    EPT = -(-E // (_NW * _C * _KI)) * (_C * _KI)
    NCH = EPT // _C
    PAD = _NW * EPT - E
    i32 = edge_index.dtype
    src = jnp.concatenate([edge_index[0], jnp.zeros((PAD,), i32)])
    dst = jnp.concatenate(
        [edge_index[1], N + (jnp.arange(PAD, dtype=i32) % (_NP - N))])
    src = src.reshape(_NW * NCH, _C)
    dst = dst.reshape(_NW * NCH, _C)
    tab = jnp.zeros((AP, H), jnp.float32).at[:A].set(embed_table)
    zrow = jnp.zeros((_C, H), jnp.float32)
    ones_blk = jnp.ones((_C, H), jnp.float32)

    x, s0 = _embed_call(N, H, R, AP)(h.reshape(N, 1), tab)

    agg = _agg_call(N, NCH)
    d0, d1 = _deg_call(N, NCH)(ones_blk, dst, zrow)
    a00, a01 = agg(x, src, dst, zrow)

    k1 = _layer_call(N, H, R, first=True, last=False)
    h1, deg, s1 = k1(x, a00, a01, d0, d1, W_self[0], W_neigh[0],
                     b_layers[0].reshape(1, H))

    a10, a11 = agg(h1, src, dst, zrow)

    k2 = _layer_call(N, H, R, first=False, last=False)
    h2, s2 = k2(h1, a10, a11, deg, W_self[1], W_neigh[1],
                b_layers[1].reshape(1, H))

    a20, a21 = agg(h2, src, dst, zrow)

    k3 = _layer_call(N, H, R, first=False, last=True)
    s3, = k3(h2, a20, a21, deg, W_self[2], W_neigh[2],
             b_layers[2].reshape(1, H))

    o = _readout_call(N, H, 4 * H)(
        s0, s1, s2, s3, Wr1, br1.reshape(1, -1), Wr2, br2.reshape(1, -1),
        Wr3, br3.reshape(1, -1))
    return o


# speedup vs baseline: 1.1632x; 1.1632x over previous
import functools

import jax
import jax.numpy as jnp
from jax import lax
from jax.experimental import pallas as pl
from jax.experimental.pallas import tpu as pltpu
from jax.experimental.pallas import tpu_sc as plsc

HIGHEST = lax.Precision.HIGHEST

_NC = 2
_NS = 16
_NW = _NC * _NS


def _f32dot(a, b):
    return jnp.dot(a, b, preferred_element_type=jnp.float32)


def _embed_call(N, H, R, AP):
    nb = N // R

    def body(h_ref, tab_ref, x_ref, s_ref):
        hv = h_ref[...]
        iot = lax.broadcasted_iota(jnp.int32, (R, AP), 1)
        oh = jnp.where(hv == iot, 1.0, 0.0).astype(jnp.float32)
        x = _f32dot(oh, tab_ref[...])
        x_ref[...] = x
        col = jnp.sum(x, axis=0, keepdims=True)

        @pl.when(pl.program_id(0) == 0)
        def _():
            s_ref[...] = col

        @pl.when(pl.program_id(0) > 0)
        def _():
            s_ref[...] += col

    return pl.pallas_call(
        body,
        grid=(nb,),
        in_specs=[
            pl.BlockSpec((R, 1), lambda i: (i, 0)),
            pl.BlockSpec((AP, H), lambda i: (0, 0)),
        ],
        out_specs=[
            pl.BlockSpec((R, H), lambda i: (i, 0)),
            pl.BlockSpec((1, H), lambda i: (0, 0)),
        ],
        out_shape=[
            jax.ShapeDtypeStruct((N, H), jnp.float32),
            jax.ShapeDtypeStruct((1, H), jnp.float32),
        ],
    )


def _layer_call(N, H, R, first, last):
    def body(*refs):
        if first:
            (h_ref, aa_ref, ab_ref, da_ref, db_ref, ws_ref, wn_ref, b_ref,
             o_ref, dg_ref, s_ref) = refs
            d = jnp.maximum(da_ref[...][:, 0:1] + db_ref[...][:, 0:1], 1.0)
            dg_ref[...] = d
        elif last:
            (h_ref, aa_ref, ab_ref, dg_ref, ws_ref, wn_ref, b_ref,
             s_ref) = refs
            d = dg_ref[...]
        else:
            (h_ref, aa_ref, ab_ref, dg_ref, ws_ref, wn_ref, b_ref,
             o_ref, s_ref) = refs
            d = dg_ref[...]
        agg = (aa_ref[...] + ab_ref[...]) / d
        hn = jnp.maximum(
            _f32dot(h_ref[...], ws_ref[...]) + _f32dot(agg, wn_ref[...])
            + b_ref[...], 0.0)
        if not last:
            o_ref[...] = hn
        col = jnp.sum(hn, axis=0, keepdims=True)

        @pl.when(pl.program_id(0) == 0)
        def _():
            s_ref[...] = col

        @pl.when(pl.program_id(0) > 0)
        def _():
            s_ref[...] += col

    in_specs = [
        pl.BlockSpec((R, H), lambda i: (i, 0)),
        pl.BlockSpec((R, H), lambda i: (i, 0)),
        pl.BlockSpec((R, H), lambda i: (i, 0)),
    ]
    if first:
        in_specs += [
            pl.BlockSpec((R, H), lambda i: (i, 0)),
            pl.BlockSpec((R, H), lambda i: (i, 0)),
        ]
    else:
        in_specs += [pl.BlockSpec((R, 1), lambda i: (i, 0))]
    in_specs += [pl.BlockSpec((H, H), lambda i: (0, 0)),
                 pl.BlockSpec((H, H), lambda i: (0, 0)),
                 pl.BlockSpec((1, H), lambda i: (0, 0))]

    out_specs, out_shape = [], []
    if not last:
        out_specs += [pl.BlockSpec((R, H), lambda i: (i, 0))]
        out_shape += [jax.ShapeDtypeStruct((N, H), jnp.float32)]
    if first:
        out_specs += [pl.BlockSpec((R, 1), lambda i: (i, 0))]
        out_shape += [jax.ShapeDtypeStruct((N, 1), jnp.float32)]
    out_specs += [pl.BlockSpec((1, H), lambda i: (0, 0))]
    out_shape += [jax.ShapeDtypeStruct((1, H), jnp.float32)]

    return pl.pallas_call(
        body, grid=(N // R,), in_specs=in_specs, out_specs=out_specs,
        out_shape=out_shape,
    )


def _readout_call(N, H, D):
    inv_n = 1.0 / N

    def body(s0, s1, s2, s3, w1, b1, w2, b2, w3, b3, o_ref):
        acc = b1[...]
        for k, s in enumerate((s0, s1, s2, s3)):
            acc = acc + _f32dot(s[...] * inv_n, w1[k * H:(k + 1) * H, :])
        o1 = jnp.maximum(acc, 0.0)
        o2 = jnp.maximum(_f32dot(o1, w2[...]) + b2[...], 0.0)
        o_ref[...] = _f32dot(o2, w3[...]) + b3[...]

    return pl.pallas_call(
        body,
        out_shape=jax.ShapeDtypeStruct((1, 1), jnp.float32),
    )


_C = 64
_KI = 16
_NP = 10016
_SC0_SHARE_PCT = 65


def _agg_call(N, NCH):
    H = 128
    KB = NCH // _KI
    RA = ((_NP // _NS) // 8) * 8
    RL = _NP - (_NS - 1) * RA
    assert KB * _KI == NCH and RL % 8 == 0 and RL >= RA

    mesh = plsc.VectorSubcoreMesh(core_axis_name="c", subcore_axis_name="s")

    def chunks(total):
        out, ofs = [], 0
        while ofs < total:
            c = min(_C, total - ofs)
            out.append((ofs, c))
            ofs += c
        return out

    out_type = tuple(
        jax.ShapeDtypeStruct((_NP, H), jnp.float32) for _ in range(2))
    scratch = [
        pltpu.VMEM((_KI, _C), jnp.int32),
        pltpu.VMEM((_KI, _C), jnp.int32),
        pltpu.VMEM((_C, H), jnp.float32),
        pltpu.VMEM((_C, H), jnp.float32),
        pltpu.VMEM_SHARED((_NP, H), jnp.float32),
        pltpu.SemaphoreType.DMA,
        pltpu.SemaphoreType.DMA,
    ]

    KB0 = (2 * KB * _SC0_SHARE_PCT + 50) // 100
    KB1 = 2 * KB - KB0

    def body(p_hbm, srcr, dstr, zrow, oa0, oa1,
             sidx, didx, rows0, rows1, accs, sem0, sem1):
        cid = lax.axis_index("c")
        sid = lax.axis_index("s")
        base = pl.multiple_of(sid * RA, 8)
        kb_loc = jnp.where(cid == 0, KB0, KB1)
        irow0 = pl.multiple_of(
            jnp.where(cid == 0, sid * (KB0 * _KI),
                      _NS * (KB0 * _KI) + sid * (KB1 * _KI)), 8)

        pltpu.sync_copy(zrow, rows0)
        for is_last, rcnt in ((False, RA), (True, RL)):
            cond = (sid == _NS - 1) if is_last else (sid < _NS - 1)

            @pl.when(cond)
            def _():
                for ofs, c in chunks(rcnt):
                    sl = pl.ds(pl.multiple_of(base + ofs, 8), c)
                    pltpu.sync_copy(rows0.at[pl.ds(0, c)], accs.at[sl])

        plsc.subcore_barrier()

        def outer(kb, carry):
            koff = pl.multiple_of(irow0 + kb * _KI, 8)
            pltpu.sync_copy(srcr.at[pl.ds(koff, _KI)], sidx)
            pltpu.sync_copy(dstr.at[pl.ds(koff, _KI)], didx)
            pltpu.make_async_copy(p_hbm.at[sidx.at[0]], rows0, sem0).start()

            def pipe(j, cur, csem, nxt, nsem):
                pltpu.make_async_copy(p_hbm.at[sidx.at[j]], cur, csem).wait()

                @pl.when(j < _KI - 1)
                def _():
                    pltpu.make_async_copy(
                        p_hbm.at[sidx.at[j + 1]], nxt, nsem).start()

                pltpu.sync_copy(cur, accs.at[didx.at[j]], add=True)

            def step(j, c2):
                even = j % 2 == 0

                @pl.when(even)
                def _():
                    pipe(j, rows0, sem0, rows1, sem1)

                @pl.when(jnp.logical_not(even))
                def _():
                    pipe(j, rows1, sem1, rows0, sem0)

                return c2

            lax.fori_loop(0, _KI, step, 0)
            return carry

        lax.fori_loop(0, kb_loc, outer, 0)
        plsc.subcore_barrier()

        for is_last, rcnt in ((False, RA), (True, RL)):
            cond = (sid == _NS - 1) if is_last else (sid < _NS - 1)

            @pl.when(cond)
            def _():
                for c0, oa in enumerate((oa0, oa1)):
                    @pl.when(cid == c0)
                    def _():
                        for ofs, c in chunks(rcnt):
                            sl = pl.ds(pl.multiple_of(base + ofs, 8), c)
                            pltpu.sync_copy(accs.at[sl], rows0.at[pl.ds(0, c)])
                            pltpu.sync_copy(rows0.at[pl.ds(0, c)], oa.at[sl])

    return pl.kernel(
        body, mesh=mesh, out_type=out_type, scratch_types=scratch,
    )


def _deg_call(N, NCH):
    H = 128
    KB = NCH // _KI
    RA = ((_NP // _NS) // 8) * 8
    RL = _NP - (_NS - 1) * RA

    mesh = plsc.VectorSubcoreMesh(core_axis_name="c", subcore_axis_name="s")

    def chunks(total):
        out, ofs = [], 0
        while ofs < total:
            c = min(_C, total - ofs)
            out.append((ofs, c))
            ofs += c
        return out

    out_type = tuple(
        jax.ShapeDtypeStruct((_NP, H), jnp.float32) for _ in range(2))
    scratch = [
        pltpu.VMEM((_KI, _C), jnp.int32),
        pltpu.VMEM((_C, H), jnp.float32),
        pltpu.VMEM_SHARED((_NP, H), jnp.float32),
    ]

    def body(ones_hbm, dstr, zrow, od0, od1, didx, rows, accs):
        cid = lax.axis_index("c")
        sid = lax.axis_index("s")
        base = pl.multiple_of(sid * RA, 8)
        irow0 = pl.multiple_of((cid * _NS + sid) * NCH, 8)

        pltpu.sync_copy(zrow, rows)
        for is_last, rcnt in ((False, RA), (True, RL)):
            cond = (sid == _NS - 1) if is_last else (sid < _NS - 1)

            @pl.when(cond)
            def _():
                for ofs, c in chunks(rcnt):
                    sl = pl.ds(pl.multiple_of(base + ofs, 8), c)
                    pltpu.sync_copy(rows.at[pl.ds(0, c)], accs.at[sl])

        pltpu.sync_copy(ones_hbm, rows)
        plsc.subcore_barrier()

        def outer(kb, carry):
            koff = pl.multiple_of(irow0 + kb * _KI, 8)
            pltpu.sync_copy(dstr.at[pl.ds(koff, _KI)], didx)

            def step(j, c2):
                pltpu.sync_copy(rows, accs.at[didx.at[j]], add=True)
                return c2

            lax.fori_loop(0, _KI, step, 0)
            return carry

        lax.fori_loop(0, KB, outer, 0)
        plsc.subcore_barrier()

        for is_last, rcnt in ((False, RA), (True, RL)):
            cond = (sid == _NS - 1) if is_last else (sid < _NS - 1)

            @pl.when(cond)
            def _():
                for c0, od in enumerate((od0, od1)):
                    @pl.when(cid == c0)
                    def _():
                        for ofs, c in chunks(rcnt):
                            sl = pl.ds(pl.multiple_of(base + ofs, 8), c)
                            pltpu.sync_copy(accs.at[sl], rows.at[pl.ds(0, c)])
                            pltpu.sync_copy(rows.at[pl.ds(0, c)], od.at[sl])

    return pl.kernel(
        body, mesh=mesh, out_type=out_type, scratch_types=scratch,
    )


def kernel(h, edge_index, e, embed_table, W_self, W_neigh, b_layers,
           Wr1, br1, Wr2, br2, Wr3, br3):
    del e
    N = h.shape[0]
    E = edge_index.shape[1]
    A, H = embed_table.shape
    AP = 128
    R = 2000

    EPT = -(-E // (_NW * _C * _KI)) * (_C * _KI)
    NCH = EPT // _C
    PAD = _NW * EPT - E
    i32 = edge_index.dtype
    src = jnp.concatenate([edge_index[0], jnp.zeros((PAD,), i32)])
    dst = jnp.concatenate(
        [edge_index[1], N + (jnp.arange(PAD, dtype=i32) % (_NP - N))])
    src = src.reshape(_NW * NCH, _C)
    dst = dst.reshape(_NW * NCH, _C)
    tab = jnp.zeros((AP, H), jnp.float32).at[:A].set(embed_table)
    zrow = jnp.zeros((_C, H), jnp.float32)
    ones_blk = jnp.ones((_C, H), jnp.float32)

    x, s0 = _embed_call(N, H, R, AP)(h.reshape(N, 1), tab)

    agg = _agg_call(N, NCH)
    d0, d1 = _deg_call(N, NCH)(ones_blk, dst, zrow)
    a00, a01 = agg(x, src, dst, zrow)

    k1 = _layer_call(N, H, R, first=True, last=False)
    h1, deg, s1 = k1(x, a00, a01, d0, d1, W_self[0], W_neigh[0],
                     b_layers[0].reshape(1, H))

    a10, a11 = agg(h1, src, dst, zrow)

    k2 = _layer_call(N, H, R, first=False, last=False)
    h2, s2 = k2(h1, a10, a11, deg, W_self[1], W_neigh[1],
                b_layers[1].reshape(1, H))

    a20, a21 = agg(h2, src, dst, zrow)

    k3 = _layer_call(N, H, R, first=False, last=True)
    s3, = k3(h2, a20, a21, deg, W_self[2], W_neigh[2],
             b_layers[2].reshape(1, H))

    o = _readout_call(N, H, 4 * H)(
        s0, s1, s2, s3, Wr1, br1.reshape(1, -1), Wr2, br2.reshape(1, -1),
        Wr3, br3.reshape(1, -1))
    return o


# speedup vs baseline: 1.1975x; 1.0295x over previous
import functools

import jax
import jax.numpy as jnp
from jax import lax
from jax.experimental import pallas as pl
from jax.experimental.pallas import tpu as pltpu
from jax.experimental.pallas import tpu_sc as plsc

HIGHEST = lax.Precision.HIGHEST

_NC = 2
_NS = 16
_NW = _NC * _NS


def _f32dot(a, b):
    return jnp.dot(a, b, preferred_element_type=jnp.float32)


def _embed_call(N, H, R, AP):
    nb = N // R

    def body(h_ref, tab_ref, x_ref, s_ref):
        hv = h_ref[...]
        iot = lax.broadcasted_iota(jnp.int32, (R, AP), 1)
        oh = jnp.where(hv == iot, 1.0, 0.0).astype(jnp.float32)
        x = _f32dot(oh, tab_ref[...])
        x_ref[...] = x
        col = jnp.sum(x, axis=0, keepdims=True)

        @pl.when(pl.program_id(0) == 0)
        def _():
            s_ref[...] = col

        @pl.when(pl.program_id(0) > 0)
        def _():
            s_ref[...] += col

    return pl.pallas_call(
        body,
        grid=(nb,),
        in_specs=[
            pl.BlockSpec((R, 1), lambda i: (i, 0)),
            pl.BlockSpec((AP, H), lambda i: (0, 0)),
        ],
        out_specs=[
            pl.BlockSpec((R, H), lambda i: (i, 0)),
            pl.BlockSpec((1, H), lambda i: (0, 0)),
        ],
        out_shape=[
            jax.ShapeDtypeStruct((N, H), jnp.float32),
            jax.ShapeDtypeStruct((1, H), jnp.float32),
        ],
    )


def _layer_call(N, H, R, first, last):
    def body(*refs):
        if first:
            (h_ref, aa_ref, ab_ref, da_ref, db_ref, ws_ref, wn_ref, b_ref,
             o_ref, dg_ref, s_ref) = refs
            d = jnp.maximum(da_ref[...][:, 0:1] + db_ref[...][:, 0:1], 1.0)
            dg_ref[...] = d
        elif last:
            (h_ref, aa_ref, ab_ref, dg_ref, ws_ref, wn_ref, b_ref,
             s_ref) = refs
            d = dg_ref[...]
        else:
            (h_ref, aa_ref, ab_ref, dg_ref, ws_ref, wn_ref, b_ref,
             o_ref, s_ref) = refs
            d = dg_ref[...]
        agg = (aa_ref[...] + ab_ref[...]) / d
        hn = jnp.maximum(
            _f32dot(h_ref[...], ws_ref[...]) + _f32dot(agg, wn_ref[...])
            + b_ref[...], 0.0)
        if not last:
            o_ref[...] = hn
        col = jnp.sum(hn, axis=0, keepdims=True)

        @pl.when(pl.program_id(0) == 0)
        def _():
            s_ref[...] = col

        @pl.when(pl.program_id(0) > 0)
        def _():
            s_ref[...] += col

    in_specs = [
        pl.BlockSpec((R, H), lambda i: (i, 0)),
        pl.BlockSpec((R, H), lambda i: (i, 0)),
        pl.BlockSpec((R, H), lambda i: (i, 0)),
    ]
    if first:
        in_specs += [
            pl.BlockSpec((R, H), lambda i: (i, 0)),
            pl.BlockSpec((R, H), lambda i: (i, 0)),
        ]
    else:
        in_specs += [pl.BlockSpec((R, 1), lambda i: (i, 0))]
    in_specs += [pl.BlockSpec((H, H), lambda i: (0, 0)),
                 pl.BlockSpec((H, H), lambda i: (0, 0)),
                 pl.BlockSpec((1, H), lambda i: (0, 0))]

    out_specs, out_shape = [], []
    if not last:
        out_specs += [pl.BlockSpec((R, H), lambda i: (i, 0))]
        out_shape += [jax.ShapeDtypeStruct((N, H), jnp.float32)]
    if first:
        out_specs += [pl.BlockSpec((R, 1), lambda i: (i, 0))]
        out_shape += [jax.ShapeDtypeStruct((N, 1), jnp.float32)]
    out_specs += [pl.BlockSpec((1, H), lambda i: (0, 0))]
    out_shape += [jax.ShapeDtypeStruct((1, H), jnp.float32)]

    return pl.pallas_call(
        body, grid=(N // R,), in_specs=in_specs, out_specs=out_specs,
        out_shape=out_shape,
    )


def _readout_call(N, H, D):
    inv_n = 1.0 / N

    def body(s0, s1, s2, s3, w1, b1, w2, b2, w3, b3, o_ref):
        acc = b1[...]
        for k, s in enumerate((s0, s1, s2, s3)):
            acc = acc + _f32dot(s[...] * inv_n, w1[k * H:(k + 1) * H, :])
        o1 = jnp.maximum(acc, 0.0)
        o2 = jnp.maximum(_f32dot(o1, w2[...]) + b2[...], 0.0)
        o_ref[...] = _f32dot(o2, w3[...]) + b3[...]

    return pl.pallas_call(
        body,
        out_shape=jax.ShapeDtypeStruct((1, 1), jnp.float32),
    )


_C = 64
_KI = 16
_NP = 10016
_SC0_SHARE_PCT = 70


def _agg_call(N, NCH):
    H = 128
    KB = NCH // _KI
    RA = ((_NP // _NS) // 8) * 8
    RL = _NP - (_NS - 1) * RA
    assert KB * _KI == NCH and RL % 8 == 0 and RL >= RA

    mesh = plsc.VectorSubcoreMesh(core_axis_name="c", subcore_axis_name="s")

    def chunks(total):
        out, ofs = [], 0
        while ofs < total:
            c = min(_C, total - ofs)
            out.append((ofs, c))
            ofs += c
        return out

    out_type = tuple(
        jax.ShapeDtypeStruct((_NP, H), jnp.float32) for _ in range(2))
    scratch = [
        pltpu.VMEM((_KI, _C), jnp.int32),
        pltpu.VMEM((_KI, _C), jnp.int32),
        pltpu.VMEM((_C, H), jnp.float32),
        pltpu.VMEM((_C, H), jnp.float32),
        pltpu.VMEM_SHARED((_NP, H), jnp.float32),
        pltpu.SemaphoreType.DMA,
        pltpu.SemaphoreType.DMA,
    ]

    KB0 = (2 * KB * _SC0_SHARE_PCT + 50) // 100
    KB1 = 2 * KB - KB0

    def body(p_hbm, srcr, dstr, zrow, oa0, oa1,
             sidx, didx, rows0, rows1, accs, sem0, sem1):
        cid = lax.axis_index("c")
        sid = lax.axis_index("s")
        base = pl.multiple_of(sid * RA, 8)
        kb_loc = jnp.where(cid == 0, KB0, KB1)
        irow0 = pl.multiple_of(
            jnp.where(cid == 0, sid * (KB0 * _KI),
                      _NS * (KB0 * _KI) + sid * (KB1 * _KI)), 8)

        pltpu.sync_copy(zrow, rows0)
        for is_last, rcnt in ((False, RA), (True, RL)):
            cond = (sid == _NS - 1) if is_last else (sid < _NS - 1)

            @pl.when(cond)
            def _():
                for ofs, c in chunks(rcnt):
                    sl = pl.ds(pl.multiple_of(base + ofs, 8), c)
                    pltpu.sync_copy(rows0.at[pl.ds(0, c)], accs.at[sl])

        plsc.subcore_barrier()

        def outer(kb, carry):
            koff = pl.multiple_of(irow0 + kb * _KI, 8)
            pltpu.sync_copy(srcr.at[pl.ds(koff, _KI)], sidx)
            pltpu.sync_copy(dstr.at[pl.ds(koff, _KI)], didx)
            pltpu.make_async_copy(p_hbm.at[sidx.at[0]], rows0, sem0).start()

            def pipe(j, cur, csem, nxt, nsem):
                pltpu.make_async_copy(p_hbm.at[sidx.at[j]], cur, csem).wait()

                @pl.when(j < _KI - 1)
                def _():
                    pltpu.make_async_copy(
                        p_hbm.at[sidx.at[j + 1]], nxt, nsem).start()

                pltpu.sync_copy(cur, accs.at[didx.at[j]], add=True)

            def step(j, c2):
                even = j % 2 == 0

                @pl.when(even)
                def _():
                    pipe(j, rows0, sem0, rows1, sem1)

                @pl.when(jnp.logical_not(even))
                def _():
                    pipe(j, rows1, sem1, rows0, sem0)

                return c2

            lax.fori_loop(0, _KI, step, 0)
            return carry

        lax.fori_loop(0, kb_loc, outer, 0)
        plsc.subcore_barrier()

        for is_last, rcnt in ((False, RA), (True, RL)):
            cond = (sid == _NS - 1) if is_last else (sid < _NS - 1)

            @pl.when(cond)
            def _():
                for c0, oa in enumerate((oa0, oa1)):
                    @pl.when(cid == c0)
                    def _():
                        for ofs, c in chunks(rcnt):
                            sl = pl.ds(pl.multiple_of(base + ofs, 8), c)
                            pltpu.sync_copy(accs.at[sl], rows0.at[pl.ds(0, c)])
                            pltpu.sync_copy(rows0.at[pl.ds(0, c)], oa.at[sl])

    return pl.kernel(
        body, mesh=mesh, out_type=out_type, scratch_types=scratch,
    )


def _deg_call(N, NCH):
    H = 128
    KB = NCH // _KI
    RA = ((_NP // _NS) // 8) * 8
    RL = _NP - (_NS - 1) * RA

    mesh = plsc.VectorSubcoreMesh(core_axis_name="c", subcore_axis_name="s")

    def chunks(total):
        out, ofs = [], 0
        while ofs < total:
            c = min(_C, total - ofs)
            out.append((ofs, c))
            ofs += c
        return out

    out_type = tuple(
        jax.ShapeDtypeStruct((_NP, H), jnp.float32) for _ in range(2))
    scratch = [
        pltpu.VMEM((_KI, _C), jnp.int32),
        pltpu.VMEM((_C, H), jnp.float32),
        pltpu.VMEM_SHARED((_NP, H), jnp.float32),
    ]

    def body(ones_hbm, dstr, zrow, od0, od1, didx, rows, accs):
        cid = lax.axis_index("c")
        sid = lax.axis_index("s")
        base = pl.multiple_of(sid * RA, 8)
        irow0 = pl.multiple_of((cid * _NS + sid) * NCH, 8)

        pltpu.sync_copy(zrow, rows)
        for is_last, rcnt in ((False, RA), (True, RL)):
            cond = (sid == _NS - 1) if is_last else (sid < _NS - 1)

            @pl.when(cond)
            def _():
                for ofs, c in chunks(rcnt):
                    sl = pl.ds(pl.multiple_of(base + ofs, 8), c)
                    pltpu.sync_copy(rows.at[pl.ds(0, c)], accs.at[sl])

        pltpu.sync_copy(ones_hbm, rows)
        plsc.subcore_barrier()

        def outer(kb, carry):
            koff = pl.multiple_of(irow0 + kb * _KI, 8)
            pltpu.sync_copy(dstr.at[pl.ds(koff, _KI)], didx)

            def step(j, c2):
                pltpu.sync_copy(rows, accs.at[didx.at[j]], add=True)
                return c2

            lax.fori_loop(0, _KI, step, 0)
            return carry

        lax.fori_loop(0, KB, outer, 0)
        plsc.subcore_barrier()

        for is_last, rcnt in ((False, RA), (True, RL)):
            cond = (sid == _NS - 1) if is_last else (sid < _NS - 1)

            @pl.when(cond)
            def _():
                for c0, od in enumerate((od0, od1)):
                    @pl.when(cid == c0)
                    def _():
                        for ofs, c in chunks(rcnt):
                            sl = pl.ds(pl.multiple_of(base + ofs, 8), c)
                            pltpu.sync_copy(accs.at[sl], rows.at[pl.ds(0, c)])
                            pltpu.sync_copy(rows.at[pl.ds(0, c)], od.at[sl])

    return pl.kernel(
        body, mesh=mesh, out_type=out_type, scratch_types=scratch,
    )


def kernel(h, edge_index, e, embed_table, W_self, W_neigh, b_layers,
           Wr1, br1, Wr2, br2, Wr3, br3):
    del e
    N = h.shape[0]
    E = edge_index.shape[1]
    A, H = embed_table.shape
    AP = 128
    R = 2000

    EPT = -(-E // (_NW * _C * _KI)) * (_C * _KI)
    NCH = EPT // _C
    PAD = _NW * EPT - E
    i32 = edge_index.dtype
    src = jnp.concatenate([edge_index[0], jnp.zeros((PAD,), i32)])
    dst = jnp.concatenate(
        [edge_index[1], N + (jnp.arange(PAD, dtype=i32) % (_NP - N))])
    src = src.reshape(_NW * NCH, _C)
    dst = dst.reshape(_NW * NCH, _C)
    tab = jnp.zeros((AP, H), jnp.float32).at[:A].set(embed_table)
    zrow = jnp.zeros((_C, H), jnp.float32)
    ones_blk = jnp.ones((_C, H), jnp.float32)

    x, s0 = _embed_call(N, H, R, AP)(h.reshape(N, 1), tab)

    agg = _agg_call(N, NCH)
    d0, d1 = _deg_call(N, NCH)(ones_blk, dst, zrow)
    a00, a01 = agg(x, src, dst, zrow)

    k1 = _layer_call(N, H, R, first=True, last=False)
    h1, deg, s1 = k1(x, a00, a01, d0, d1, W_self[0], W_neigh[0],
                     b_layers[0].reshape(1, H))

    a10, a11 = agg(h1, src, dst, zrow)

    k2 = _layer_call(N, H, R, first=False, last=False)
    h2, s2 = k2(h1, a10, a11, deg, W_self[1], W_neigh[1],
                b_layers[1].reshape(1, H))

    a20, a21 = agg(h2, src, dst, zrow)

    k3 = _layer_call(N, H, R, first=False, last=True)
    s3, = k3(h2, a20, a21, deg, W_self[2], W_neigh[2],
             b_layers[2].reshape(1, H))

    o = _readout_call(N, H, 4 * H)(
        s0, s1, s2, s3, Wr1, br1.reshape(1, -1), Wr2, br2.reshape(1, -1),
        Wr3, br3.reshape(1, -1))
    return o


# speedup vs baseline: 1.2303x; 1.0275x over previous
import functools

import jax
import jax.numpy as jnp
from jax import lax
from jax.experimental import pallas as pl
from jax.experimental.pallas import tpu as pltpu
from jax.experimental.pallas import tpu_sc as plsc

HIGHEST = lax.Precision.HIGHEST

_NC = 2
_NS = 16
_NW = _NC * _NS


def _f32dot(a, b):
    return jnp.dot(a, b, preferred_element_type=jnp.float32)


def _embed_call(N, H, R, AP):
    nb = N // R

    def body(h_ref, tab_ref, x_ref, s_ref):
        hv = h_ref[...]
        iot = lax.broadcasted_iota(jnp.int32, (R, AP), 1)
        oh = jnp.where(hv == iot, 1.0, 0.0).astype(jnp.float32)
        x = _f32dot(oh, tab_ref[...])
        x_ref[...] = x
        col = jnp.sum(x, axis=0, keepdims=True)

        @pl.when(pl.program_id(0) == 0)
        def _():
            s_ref[...] = col

        @pl.when(pl.program_id(0) > 0)
        def _():
            s_ref[...] += col

    return pl.pallas_call(
        body,
        grid=(nb,),
        in_specs=[
            pl.BlockSpec((R, 1), lambda i: (i, 0)),
            pl.BlockSpec((AP, H), lambda i: (0, 0)),
        ],
        out_specs=[
            pl.BlockSpec((R, H), lambda i: (i, 0)),
            pl.BlockSpec((1, H), lambda i: (0, 0)),
        ],
        out_shape=[
            jax.ShapeDtypeStruct((N, H), jnp.float32),
            jax.ShapeDtypeStruct((1, H), jnp.float32),
        ],
    )


def _layer_call(N, H, R, first, last):
    def body(*refs):
        if first:
            (h_ref, aa_ref, ab_ref, da_ref, db_ref, ws_ref, wn_ref, b_ref,
             o_ref, dg_ref, s_ref) = refs
            d = jnp.maximum(da_ref[...][:, 0:1] + db_ref[...][:, 0:1], 1.0)
            dg_ref[...] = d
        elif last:
            (h_ref, aa_ref, ab_ref, dg_ref, ws_ref, wn_ref, b_ref,
             s_ref) = refs
            d = dg_ref[...]
        else:
            (h_ref, aa_ref, ab_ref, dg_ref, ws_ref, wn_ref, b_ref,
             o_ref, s_ref) = refs
            d = dg_ref[...]
        agg = (aa_ref[...] + ab_ref[...]) / d
        hn = jnp.maximum(
            _f32dot(h_ref[...], ws_ref[...]) + _f32dot(agg, wn_ref[...])
            + b_ref[...], 0.0)
        if not last:
            o_ref[...] = hn
        col = jnp.sum(hn, axis=0, keepdims=True)

        @pl.when(pl.program_id(0) == 0)
        def _():
            s_ref[...] = col

        @pl.when(pl.program_id(0) > 0)
        def _():
            s_ref[...] += col

    in_specs = [
        pl.BlockSpec((R, H), lambda i: (i, 0)),
        pl.BlockSpec((R, H), lambda i: (i, 0)),
        pl.BlockSpec((R, H), lambda i: (i, 0)),
    ]
    if first:
        in_specs += [
            pl.BlockSpec((R, H), lambda i: (i, 0)),
            pl.BlockSpec((R, H), lambda i: (i, 0)),
        ]
    else:
        in_specs += [pl.BlockSpec((R, 1), lambda i: (i, 0))]
    in_specs += [pl.BlockSpec((H, H), lambda i: (0, 0)),
                 pl.BlockSpec((H, H), lambda i: (0, 0)),
                 pl.BlockSpec((1, H), lambda i: (0, 0))]

    out_specs, out_shape = [], []
    if not last:
        out_specs += [pl.BlockSpec((R, H), lambda i: (i, 0))]
        out_shape += [jax.ShapeDtypeStruct((N, H), jnp.float32)]
    if first:
        out_specs += [pl.BlockSpec((R, 1), lambda i: (i, 0))]
        out_shape += [jax.ShapeDtypeStruct((N, 1), jnp.float32)]
    out_specs += [pl.BlockSpec((1, H), lambda i: (0, 0))]
    out_shape += [jax.ShapeDtypeStruct((1, H), jnp.float32)]

    return pl.pallas_call(
        body, grid=(N // R,), in_specs=in_specs, out_specs=out_specs,
        out_shape=out_shape,
    )


def _readout_call(N, H, D):
    inv_n = 1.0 / N

    def body(s0, s1, s2, s3, w1, b1, w2, b2, w3, b3, o_ref):
        acc = b1[...]
        for k, s in enumerate((s0, s1, s2, s3)):
            acc = acc + _f32dot(s[...] * inv_n, w1[k * H:(k + 1) * H, :])
        o1 = jnp.maximum(acc, 0.0)
        o2 = jnp.maximum(_f32dot(o1, w2[...]) + b2[...], 0.0)
        o_ref[...] = _f32dot(o2, w3[...]) + b3[...]

    return pl.pallas_call(
        body,
        out_shape=jax.ShapeDtypeStruct((1, 1), jnp.float32),
    )


_C = 64
_KI = 16
_NP = 10016
_SC0_SHARE_PCT = 75


def _agg_call(N, NCH):
    H = 128
    KB = NCH // _KI
    RA = ((_NP // _NS) // 8) * 8
    RL = _NP - (_NS - 1) * RA
    assert KB * _KI == NCH and RL % 8 == 0 and RL >= RA

    mesh = plsc.VectorSubcoreMesh(core_axis_name="c", subcore_axis_name="s")

    def chunks(total):
        out, ofs = [], 0
        while ofs < total:
            c = min(_C, total - ofs)
            out.append((ofs, c))
            ofs += c
        return out

    out_type = tuple(
        jax.ShapeDtypeStruct((_NP, H), jnp.float32) for _ in range(2))
    scratch = [
        pltpu.VMEM((_KI, _C), jnp.int32),
        pltpu.VMEM((_KI, _C), jnp.int32),
        pltpu.VMEM((_C, H), jnp.float32),
        pltpu.VMEM((_C, H), jnp.float32),
        pltpu.VMEM_SHARED((_NP, H), jnp.float32),
        pltpu.SemaphoreType.DMA,
        pltpu.SemaphoreType.DMA,
    ]

    KB0 = (2 * KB * _SC0_SHARE_PCT + 50) // 100
    KB1 = 2 * KB - KB0

    def body(p_hbm, srcr, dstr, zrow, oa0, oa1,
             sidx, didx, rows0, rows1, accs, sem0, sem1):
        cid = lax.axis_index("c")
        sid = lax.axis_index("s")
        base = pl.multiple_of(sid * RA, 8)
        kb_loc = jnp.where(cid == 0, KB0, KB1)
        irow0 = pl.multiple_of(
            jnp.where(cid == 0, sid * (KB0 * _KI),
                      _NS * (KB0 * _KI) + sid * (KB1 * _KI)), 8)

        pltpu.sync_copy(zrow, rows0)
        for is_last, rcnt in ((False, RA), (True, RL)):
            cond = (sid == _NS - 1) if is_last else (sid < _NS - 1)

            @pl.when(cond)
            def _():
                for ofs, c in chunks(rcnt):
                    sl = pl.ds(pl.multiple_of(base + ofs, 8), c)
                    pltpu.sync_copy(rows0.at[pl.ds(0, c)], accs.at[sl])

        plsc.subcore_barrier()

        def outer(kb, carry):
            koff = pl.multiple_of(irow0 + kb * _KI, 8)
            pltpu.sync_copy(srcr.at[pl.ds(koff, _KI)], sidx)
            pltpu.sync_copy(dstr.at[pl.ds(koff, _KI)], didx)
            pltpu.make_async_copy(p_hbm.at[sidx.at[0]], rows0, sem0).start()

            def pipe(j, cur, csem, nxt, nsem):
                pltpu.make_async_copy(p_hbm.at[sidx.at[j]], cur, csem).wait()

                @pl.when(j < _KI - 1)
                def _():
                    pltpu.make_async_copy(
                        p_hbm.at[sidx.at[j + 1]], nxt, nsem).start()

                pltpu.sync_copy(cur, accs.at[didx.at[j]], add=True)

            def step(j, c2):
                even = j % 2 == 0

                @pl.when(even)
                def _():
                    pipe(j, rows0, sem0, rows1, sem1)

                @pl.when(jnp.logical_not(even))
                def _():
                    pipe(j, rows1, sem1, rows0, sem0)

                return c2

            lax.fori_loop(0, _KI, step, 0)
            return carry

        lax.fori_loop(0, kb_loc, outer, 0)
        plsc.subcore_barrier()

        for is_last, rcnt in ((False, RA), (True, RL)):
            cond = (sid == _NS - 1) if is_last else (sid < _NS - 1)

            @pl.when(cond)
            def _():
                for c0, oa in enumerate((oa0, oa1)):
                    @pl.when(cid == c0)
                    def _():
                        for ofs, c in chunks(rcnt):
                            sl = pl.ds(pl.multiple_of(base + ofs, 8), c)
                            pltpu.sync_copy(accs.at[sl], rows0.at[pl.ds(0, c)])
                            pltpu.sync_copy(rows0.at[pl.ds(0, c)], oa.at[sl])

    return pl.kernel(
        body, mesh=mesh, out_type=out_type, scratch_types=scratch,
    )


def _deg_call(N, NCH):
    H = 128
    KB = NCH // _KI
    RA = ((_NP // _NS) // 8) * 8
    RL = _NP - (_NS - 1) * RA

    mesh = plsc.VectorSubcoreMesh(core_axis_name="c", subcore_axis_name="s")

    def chunks(total):
        out, ofs = [], 0
        while ofs < total:
            c = min(_C, total - ofs)
            out.append((ofs, c))
            ofs += c
        return out

    out_type = tuple(
        jax.ShapeDtypeStruct((_NP, H), jnp.float32) for _ in range(2))
    scratch = [
        pltpu.VMEM((_KI, _C), jnp.int32),
        pltpu.VMEM((_C, H), jnp.float32),
        pltpu.VMEM_SHARED((_NP, H), jnp.float32),
    ]

    def body(ones_hbm, dstr, zrow, od0, od1, didx, rows, accs):
        cid = lax.axis_index("c")
        sid = lax.axis_index("s")
        base = pl.multiple_of(sid * RA, 8)
        irow0 = pl.multiple_of((cid * _NS + sid) * NCH, 8)

        pltpu.sync_copy(zrow, rows)
        for is_last, rcnt in ((False, RA), (True, RL)):
            cond = (sid == _NS - 1) if is_last else (sid < _NS - 1)

            @pl.when(cond)
            def _():
                for ofs, c in chunks(rcnt):
                    sl = pl.ds(pl.multiple_of(base + ofs, 8), c)
                    pltpu.sync_copy(rows.at[pl.ds(0, c)], accs.at[sl])

        pltpu.sync_copy(ones_hbm, rows)
        plsc.subcore_barrier()

        def outer(kb, carry):
            koff = pl.multiple_of(irow0 + kb * _KI, 8)
            pltpu.sync_copy(dstr.at[pl.ds(koff, _KI)], didx)

            def step(j, c2):
                pltpu.sync_copy(rows, accs.at[didx.at[j]], add=True)
                return c2

            lax.fori_loop(0, _KI, step, 0)
            return carry

        lax.fori_loop(0, KB, outer, 0)
        plsc.subcore_barrier()

        for is_last, rcnt in ((False, RA), (True, RL)):
            cond = (sid == _NS - 1) if is_last else (sid < _NS - 1)

            @pl.when(cond)
            def _():
                for c0, od in enumerate((od0, od1)):
                    @pl.when(cid == c0)
                    def _():
                        for ofs, c in chunks(rcnt):
                            sl = pl.ds(pl.multiple_of(base + ofs, 8), c)
                            pltpu.sync_copy(accs.at[sl], rows.at[pl.ds(0, c)])
                            pltpu.sync_copy(rows.at[pl.ds(0, c)], od.at[sl])

    return pl.kernel(
        body, mesh=mesh, out_type=out_type, scratch_types=scratch,
    )


def kernel(h, edge_index, e, embed_table, W_self, W_neigh, b_layers,
           Wr1, br1, Wr2, br2, Wr3, br3):
    del e
    N = h.shape[0]
    E = edge_index.shape[1]
    A, H = embed_table.shape
    AP = 128
    R = 2000

    EPT = -(-E // (_NW * _C * _KI)) * (_C * _KI)
    NCH = EPT // _C
    PAD = _NW * EPT - E
    i32 = edge_index.dtype
    src = jnp.concatenate([edge_index[0], jnp.zeros((PAD,), i32)])
    dst = jnp.concatenate(
        [edge_index[1], N + (jnp.arange(PAD, dtype=i32) % (_NP - N))])
    src = src.reshape(_NW * NCH, _C)
    dst = dst.reshape(_NW * NCH, _C)
    tab = jnp.zeros((AP, H), jnp.float32).at[:A].set(embed_table)
    zrow = jnp.zeros((_C, H), jnp.float32)
    ones_blk = jnp.ones((_C, H), jnp.float32)

    x, s0 = _embed_call(N, H, R, AP)(h.reshape(N, 1), tab)

    agg = _agg_call(N, NCH)
    d0, d1 = _deg_call(N, NCH)(ones_blk, dst, zrow)
    a00, a01 = agg(x, src, dst, zrow)

    k1 = _layer_call(N, H, R, first=True, last=False)
    h1, deg, s1 = k1(x, a00, a01, d0, d1, W_self[0], W_neigh[0],
                     b_layers[0].reshape(1, H))

    a10, a11 = agg(h1, src, dst, zrow)

    k2 = _layer_call(N, H, R, first=False, last=False)
    h2, s2 = k2(h1, a10, a11, deg, W_self[1], W_neigh[1],
                b_layers[1].reshape(1, H))

    a20, a21 = agg(h2, src, dst, zrow)

    k3 = _layer_call(N, H, R, first=False, last=True)
    s3, = k3(h2, a20, a21, deg, W_self[2], W_neigh[2],
             b_layers[2].reshape(1, H))

    o = _readout_call(N, H, 4 * H)(
        s0, s1, s2, s3, Wr1, br1.reshape(1, -1), Wr2, br2.reshape(1, -1),
        Wr3, br3.reshape(1, -1))
    return o


# speedup vs baseline: 1.2680x; 1.0306x over previous
import functools

import jax
import jax.numpy as jnp
from jax import lax
from jax.experimental import pallas as pl
from jax.experimental.pallas import tpu as pltpu
from jax.experimental.pallas import tpu_sc as plsc

HIGHEST = lax.Precision.HIGHEST

_NC = 2
_NS = 16
_NW = _NC * _NS


def _f32dot(a, b):
    return jnp.dot(a, b, preferred_element_type=jnp.float32)


def _embed_call(N, H, R, AP):
    nb = N // R

    def body(h_ref, tab_ref, x_ref, s_ref):
        hv = h_ref[...]
        iot = lax.broadcasted_iota(jnp.int32, (R, AP), 1)
        oh = jnp.where(hv == iot, 1.0, 0.0).astype(jnp.float32)
        x = _f32dot(oh, tab_ref[...])
        x_ref[...] = x
        col = jnp.sum(x, axis=0, keepdims=True)

        @pl.when(pl.program_id(0) == 0)
        def _():
            s_ref[...] = col

        @pl.when(pl.program_id(0) > 0)
        def _():
            s_ref[...] += col

    return pl.pallas_call(
        body,
        grid=(nb,),
        in_specs=[
            pl.BlockSpec((R, 1), lambda i: (i, 0)),
            pl.BlockSpec((AP, H), lambda i: (0, 0)),
        ],
        out_specs=[
            pl.BlockSpec((R, H), lambda i: (i, 0)),
            pl.BlockSpec((1, H), lambda i: (0, 0)),
        ],
        out_shape=[
            jax.ShapeDtypeStruct((N, H), jnp.float32),
            jax.ShapeDtypeStruct((1, H), jnp.float32),
        ],
    )


def _layer_call(N, H, R, first, last):
    def body(*refs):
        if first:
            (h_ref, aa_ref, ab_ref, da_ref, db_ref, ws_ref, wn_ref, b_ref,
             o_ref, dg_ref, s_ref) = refs
            d = jnp.maximum(da_ref[...][:, 0:1] + db_ref[...][:, 0:1], 1.0)
            dg_ref[...] = d
        elif last:
            (h_ref, aa_ref, ab_ref, dg_ref, ws_ref, wn_ref, b_ref,
             s_ref) = refs
            d = dg_ref[...]
        else:
            (h_ref, aa_ref, ab_ref, dg_ref, ws_ref, wn_ref, b_ref,
             o_ref, s_ref) = refs
            d = dg_ref[...]
        agg = (aa_ref[...] + ab_ref[...]) / d
        hn = jnp.maximum(
            _f32dot(h_ref[...], ws_ref[...]) + _f32dot(agg, wn_ref[...])
            + b_ref[...], 0.0)
        if not last:
            o_ref[...] = hn
        col = jnp.sum(hn, axis=0, keepdims=True)

        @pl.when(pl.program_id(0) == 0)
        def _():
            s_ref[...] = col

        @pl.when(pl.program_id(0) > 0)
        def _():
            s_ref[...] += col

    in_specs = [
        pl.BlockSpec((R, H), lambda i: (i, 0)),
        pl.BlockSpec((R, H), lambda i: (i, 0)),
        pl.BlockSpec((R, H), lambda i: (i, 0)),
    ]
    if first:
        in_specs += [
            pl.BlockSpec((R, H), lambda i: (i, 0)),
            pl.BlockSpec((R, H), lambda i: (i, 0)),
        ]
    else:
        in_specs += [pl.BlockSpec((R, 1), lambda i: (i, 0))]
    in_specs += [pl.BlockSpec((H, H), lambda i: (0, 0)),
                 pl.BlockSpec((H, H), lambda i: (0, 0)),
                 pl.BlockSpec((1, H), lambda i: (0, 0))]

    out_specs, out_shape = [], []
    if not last:
        out_specs += [pl.BlockSpec((R, H), lambda i: (i, 0))]
        out_shape += [jax.ShapeDtypeStruct((N, H), jnp.float32)]
    if first:
        out_specs += [pl.BlockSpec((R, 1), lambda i: (i, 0))]
        out_shape += [jax.ShapeDtypeStruct((N, 1), jnp.float32)]
    out_specs += [pl.BlockSpec((1, H), lambda i: (0, 0))]
    out_shape += [jax.ShapeDtypeStruct((1, H), jnp.float32)]

    return pl.pallas_call(
        body, grid=(N // R,), in_specs=in_specs, out_specs=out_specs,
        out_shape=out_shape,
    )


def _readout_call(N, H, D):
    inv_n = 1.0 / N

    def body(s0, s1, s2, s3, w1, b1, w2, b2, w3, b3, o_ref):
        acc = b1[...]
        for k, s in enumerate((s0, s1, s2, s3)):
            acc = acc + _f32dot(s[...] * inv_n, w1[k * H:(k + 1) * H, :])
        o1 = jnp.maximum(acc, 0.0)
        o2 = jnp.maximum(_f32dot(o1, w2[...]) + b2[...], 0.0)
        o_ref[...] = _f32dot(o2, w3[...]) + b3[...]

    return pl.pallas_call(
        body,
        out_shape=jax.ShapeDtypeStruct((1, 1), jnp.float32),
    )


_C = 64
_KI = 16
_NP = 10016
_SC0_SHARE_PCT = 80


def _agg_call(N, NCH):
    H = 128
    KB = NCH // _KI
    RA = ((_NP // _NS) // 8) * 8
    RL = _NP - (_NS - 1) * RA
    assert KB * _KI == NCH and RL % 8 == 0 and RL >= RA

    mesh = plsc.VectorSubcoreMesh(core_axis_name="c", subcore_axis_name="s")

    def chunks(total):
        out, ofs = [], 0
        while ofs < total:
            c = min(_C, total - ofs)
            out.append((ofs, c))
            ofs += c
        return out

    out_type = tuple(
        jax.ShapeDtypeStruct((_NP, H), jnp.float32) for _ in range(2))
    scratch = [
        pltpu.VMEM((_KI, _C), jnp.int32),
        pltpu.VMEM((_KI, _C), jnp.int32),
        pltpu.VMEM((_C, H), jnp.float32),
        pltpu.VMEM((_C, H), jnp.float32),
        pltpu.VMEM_SHARED((_NP, H), jnp.float32),
        pltpu.SemaphoreType.DMA,
        pltpu.SemaphoreType.DMA,
    ]

    KB0 = (2 * KB * _SC0_SHARE_PCT + 50) // 100
    KB1 = 2 * KB - KB0

    def body(p_hbm, srcr, dstr, zrow, oa0, oa1,
             sidx, didx, rows0, rows1, accs, sem0, sem1):
        cid = lax.axis_index("c")
        sid = lax.axis_index("s")
        base = pl.multiple_of(sid * RA, 8)
        kb_loc = jnp.where(cid == 0, KB0, KB1)
        irow0 = pl.multiple_of(
            jnp.where(cid == 0, sid * (KB0 * _KI),
                      _NS * (KB0 * _KI) + sid * (KB1 * _KI)), 8)

        pltpu.sync_copy(zrow, rows0)
        for is_last, rcnt in ((False, RA), (True, RL)):
            cond = (sid == _NS - 1) if is_last else (sid < _NS - 1)

            @pl.when(cond)
            def _():
                for ofs, c in chunks(rcnt):
                    sl = pl.ds(pl.multiple_of(base + ofs, 8), c)
                    pltpu.sync_copy(rows0.at[pl.ds(0, c)], accs.at[sl])

        plsc.subcore_barrier()

        def outer(kb, carry):
            koff = pl.multiple_of(irow0 + kb * _KI, 8)
            pltpu.sync_copy(srcr.at[pl.ds(koff, _KI)], sidx)
            pltpu.sync_copy(dstr.at[pl.ds(koff, _KI)], didx)
            pltpu.make_async_copy(p_hbm.at[sidx.at[0]], rows0, sem0).start()

            def pipe(j, cur, csem, nxt, nsem):
                pltpu.make_async_copy(p_hbm.at[sidx.at[j]], cur, csem).wait()

                @pl.when(j < _KI - 1)
                def _():
                    pltpu.make_async_copy(
                        p_hbm.at[sidx.at[j + 1]], nxt, nsem).start()

                pltpu.sync_copy(cur, accs.at[didx.at[j]], add=True)

            def step(j, c2):
                even = j % 2 == 0

                @pl.when(even)
                def _():
                    pipe(j, rows0, sem0, rows1, sem1)

                @pl.when(jnp.logical_not(even))
                def _():
                    pipe(j, rows1, sem1, rows0, sem0)

                return c2

            lax.fori_loop(0, _KI, step, 0)
            return carry

        lax.fori_loop(0, kb_loc, outer, 0)
        plsc.subcore_barrier()

        for is_last, rcnt in ((False, RA), (True, RL)):
            cond = (sid == _NS - 1) if is_last else (sid < _NS - 1)

            @pl.when(cond)
            def _():
                for c0, oa in enumerate((oa0, oa1)):
                    @pl.when(cid == c0)
                    def _():
                        for ofs, c in chunks(rcnt):
                            sl = pl.ds(pl.multiple_of(base + ofs, 8), c)
                            pltpu.sync_copy(accs.at[sl], rows0.at[pl.ds(0, c)])
                            pltpu.sync_copy(rows0.at[pl.ds(0, c)], oa.at[sl])

    return pl.kernel(
        body, mesh=mesh, out_type=out_type, scratch_types=scratch,
    )


def _deg_call(N, NCH):
    H = 128
    KB = NCH // _KI
    RA = ((_NP // _NS) // 8) * 8
    RL = _NP - (_NS - 1) * RA

    mesh = plsc.VectorSubcoreMesh(core_axis_name="c", subcore_axis_name="s")

    def chunks(total):
        out, ofs = [], 0
        while ofs < total:
            c = min(_C, total - ofs)
            out.append((ofs, c))
            ofs += c
        return out

    out_type = tuple(
        jax.ShapeDtypeStruct((_NP, H), jnp.float32) for _ in range(2))
    scratch = [
        pltpu.VMEM((_KI, _C), jnp.int32),
        pltpu.VMEM((_C, H), jnp.float32),
        pltpu.VMEM_SHARED((_NP, H), jnp.float32),
    ]

    def body(ones_hbm, dstr, zrow, od0, od1, didx, rows, accs):
        cid = lax.axis_index("c")
        sid = lax.axis_index("s")
        base = pl.multiple_of(sid * RA, 8)
        irow0 = pl.multiple_of((cid * _NS + sid) * NCH, 8)

        pltpu.sync_copy(zrow, rows)
        for is_last, rcnt in ((False, RA), (True, RL)):
            cond = (sid == _NS - 1) if is_last else (sid < _NS - 1)

            @pl.when(cond)
            def _():
                for ofs, c in chunks(rcnt):
                    sl = pl.ds(pl.multiple_of(base + ofs, 8), c)
                    pltpu.sync_copy(rows.at[pl.ds(0, c)], accs.at[sl])

        pltpu.sync_copy(ones_hbm, rows)
        plsc.subcore_barrier()

        def outer(kb, carry):
            koff = pl.multiple_of(irow0 + kb * _KI, 8)
            pltpu.sync_copy(dstr.at[pl.ds(koff, _KI)], didx)

            def step(j, c2):
                pltpu.sync_copy(rows, accs.at[didx.at[j]], add=True)
                return c2

            lax.fori_loop(0, _KI, step, 0)
            return carry

        lax.fori_loop(0, KB, outer, 0)
        plsc.subcore_barrier()

        for is_last, rcnt in ((False, RA), (True, RL)):
            cond = (sid == _NS - 1) if is_last else (sid < _NS - 1)

            @pl.when(cond)
            def _():
                for c0, od in enumerate((od0, od1)):
                    @pl.when(cid == c0)
                    def _():
                        for ofs, c in chunks(rcnt):
                            sl = pl.ds(pl.multiple_of(base + ofs, 8), c)
                            pltpu.sync_copy(accs.at[sl], rows.at[pl.ds(0, c)])
                            pltpu.sync_copy(rows.at[pl.ds(0, c)], od.at[sl])

    return pl.kernel(
        body, mesh=mesh, out_type=out_type, scratch_types=scratch,
    )


def kernel(h, edge_index, e, embed_table, W_self, W_neigh, b_layers,
           Wr1, br1, Wr2, br2, Wr3, br3):
    del e
    N = h.shape[0]
    E = edge_index.shape[1]
    A, H = embed_table.shape
    AP = 128
    R = 2000

    EPT = -(-E // (_NW * _C * _KI)) * (_C * _KI)
    NCH = EPT // _C
    PAD = _NW * EPT - E
    i32 = edge_index.dtype
    src = jnp.concatenate([edge_index[0], jnp.zeros((PAD,), i32)])
    dst = jnp.concatenate(
        [edge_index[1], N + (jnp.arange(PAD, dtype=i32) % (_NP - N))])
    src = src.reshape(_NW * NCH, _C)
    dst = dst.reshape(_NW * NCH, _C)
    tab = jnp.zeros((AP, H), jnp.float32).at[:A].set(embed_table)
    zrow = jnp.zeros((_C, H), jnp.float32)
    ones_blk = jnp.ones((_C, H), jnp.float32)

    x, s0 = _embed_call(N, H, R, AP)(h.reshape(N, 1), tab)

    agg = _agg_call(N, NCH)
    d0, d1 = _deg_call(N, NCH)(ones_blk, dst, zrow)
    a00, a01 = agg(x, src, dst, zrow)

    k1 = _layer_call(N, H, R, first=True, last=False)
    h1, deg, s1 = k1(x, a00, a01, d0, d1, W_self[0], W_neigh[0],
                     b_layers[0].reshape(1, H))

    a10, a11 = agg(h1, src, dst, zrow)

    k2 = _layer_call(N, H, R, first=False, last=False)
    h2, s2 = k2(h1, a10, a11, deg, W_self[1], W_neigh[1],
                b_layers[1].reshape(1, H))

    a20, a21 = agg(h2, src, dst, zrow)

    k3 = _layer_call(N, H, R, first=False, last=True)
    s3, = k3(h2, a20, a21, deg, W_self[2], W_neigh[2],
             b_layers[2].reshape(1, H))

    o = _readout_call(N, H, 4 * H)(
        s0, s1, s2, s3, Wr1, br1.reshape(1, -1), Wr2, br2.reshape(1, -1),
        Wr3, br3.reshape(1, -1))
    return o
